# baseline (speedup 1.0000x reference)
import functools

import jax
import jax.numpy as jnp
from jax import lax
from jax.experimental import pallas as pl
from jax.experimental.pallas import tpu as pltpu
from jax.experimental.pallas import tpu_sc as plsc

VOCAB = 100000
EMBED = 64
HIDDEN = 256
BATCH = 16
SEQ = 50
NTOK = BATCH * SEQ
NTOK_PAD = 1024
FC_TILE = 1024
FC_GRID = (VOCAB + FC_TILE - 1) // FC_TILE

_SC_CORES = 2
_SC_SUBCORES = 16
_SC_WORKERS = _SC_CORES * _SC_SUBCORES
_SC_CHUNK = NTOK_PAD // _SC_WORKERS


def _sc_gather(emb2, idx_half):
    mesh = plsc.VectorSubcoreMesh(core_axis_name="c", subcore_axis_name="s")

    @functools.partial(
        pl.kernel,
        mesh=mesh,
        out_type=jax.ShapeDtypeStruct((NTOK_PAD, 2 * EMBED), jnp.float32),
        scratch_types=[
            pltpu.VMEM((_SC_CHUNK,), jnp.int32),
            pltpu.VMEM((_SC_CHUNK, 2 * EMBED), jnp.float32),
            pltpu.SemaphoreType.DMA,
        ],
    )
    def gather_kernel(table_hbm, idx_hbm, out_hbm, idx_v, rows_v, sem):
        wid = lax.axis_index("s") * _SC_CORES + lax.axis_index("c")
        base = wid * _SC_CHUNK
        pltpu.sync_copy(idx_hbm.at[pl.ds(base, _SC_CHUNK)], idx_v)
        pltpu.async_copy(table_hbm.at[idx_v], rows_v, sem).wait()
        pltpu.sync_copy(rows_v, out_hbm.at[pl.ds(base, _SC_CHUNK)])

    return gather_kernel(emb2, idx_half)


def _lstm_body(e2_ref, par_ref, wih0_ref, whh0_ref, b0_ref, wih1_ref,
               whh1_ref, b1_ref, h0_ref, c0_ref, y_ref, hn_ref, cn_ref,
               g_ref, y0_ref):
    hi = jax.lax.Precision.HIGHEST

    def run_layer(gates_all, whh_ref, h_init, c_init, out_ref):
        def step(t, carry):
            h, c = carry
            gates = g_ref[pl.ds(t * BATCH, BATCH), :] + jnp.dot(
                h, whh_ref[...], preferred_element_type=jnp.float32,
                precision=hi)
            ig = jax.nn.sigmoid(gates[:, 0 * HIDDEN:1 * HIDDEN])
            fg = jax.nn.sigmoid(gates[:, 1 * HIDDEN:2 * HIDDEN])
            gg = jnp.tanh(gates[:, 2 * HIDDEN:3 * HIDDEN])
            og = jax.nn.sigmoid(gates[:, 3 * HIDDEN:4 * HIDDEN])
            c_new = fg * c + ig * gg
            h_new = og * jnp.tanh(c_new)
            out_ref[pl.ds(t * BATCH, BATCH), :] = h_new
            return h_new, c_new

        g_ref[...] = gates_all
        return lax.fori_loop(0, SEQ, step, (h_init, c_init))

    e2 = e2_ref[pl.ds(0, NTOK), :]
    odd = par_ref[...] > 0
    e = jnp.where(odd, e2[:, EMBED:], e2[:, :EMBED])

    gates0 = jnp.dot(e, wih0_ref[...], preferred_element_type=jnp.float32,
                     precision=hi) + b0_ref[...]
    h, c = run_layer(gates0, whh0_ref, h0_ref[0], c0_ref[0], y0_ref)
    hn_ref[0] = h
    cn_ref[0] = c

    gates1 = jnp.dot(y0_ref[...], wih1_ref[...],
                     preferred_element_type=jnp.float32,
                     precision=hi) + b1_ref[...]
    h, c = run_layer(gates1, whh1_ref, h0_ref[1], c0_ref[1], y_ref)
    hn_ref[1] = h
    cn_ref[1] = c


def _lstm_call(e2_pad, par, wih0t, whh0t, b0, wih1t, whh1t, b1, h0, c0):
    return pl.pallas_call(
        _lstm_body,
        out_shape=(
            jax.ShapeDtypeStruct((NTOK, HIDDEN), jnp.float32),
            jax.ShapeDtypeStruct((2, BATCH, HIDDEN), jnp.float32),
            jax.ShapeDtypeStruct((2, BATCH, HIDDEN), jnp.float32),
        ),
        scratch_shapes=[
            pltpu.VMEM((NTOK, 4 * HIDDEN), jnp.float32),
            pltpu.VMEM((NTOK, HIDDEN), jnp.float32),
        ],
    )(e2_pad, par, wih0t, whh0t, b0, wih1t, whh1t, b1, h0, c0)


def _fc_body(y_ref, w_ref, b_ref, o_ref):
    o_ref[...] = lax.dot_general(
        y_ref[...], w_ref[...], (((1,), (1,)), ((), ())),
        preferred_element_type=jnp.float32) + b_ref[...]


def _fc_call(y_bmaj, fc_w, fc_b2d):
    return pl.pallas_call(
        _fc_body,
        grid=(FC_GRID,),
        in_specs=[
            pl.BlockSpec((NTOK, HIDDEN), lambda i: (0, 0)),
            pl.BlockSpec((FC_TILE, HIDDEN), lambda i: (i, 0)),
            pl.BlockSpec((1, FC_TILE), lambda i: (0, i)),
        ],
        out_specs=pl.BlockSpec((NTOK, FC_TILE), lambda i: (0, i)),
        out_shape=jax.ShapeDtypeStruct((NTOK, VOCAB), jnp.float32),
        compiler_params=pltpu.CompilerParams(
            dimension_semantics=("parallel",)),
    )(y_bmaj, fc_w, fc_b2d)


def kernel(x, h0, c0, emb, w_ih0, w_hh0, b_ih0, b_hh0, w_ih1, w_hh1, b_ih1,
           b_hh1, fc_w, fc_b):
    idx = jnp.transpose(x).reshape(-1)
    idx_pad = jnp.concatenate(
        [idx, jnp.zeros((NTOK_PAD - NTOK,), jnp.int32)])

    emb2 = emb.reshape(VOCAB // 2, 2 * EMBED)
    e2_pad = _sc_gather(emb2, idx_pad >> 1)
    par = (idx & 1).reshape(NTOK, 1)

    wih0t = jnp.transpose(w_ih0)
    whh0t = jnp.transpose(w_hh0)
    wih1t = jnp.transpose(w_ih1)
    whh1t = jnp.transpose(w_hh1)
    b0 = (b_ih0 + b_hh0).reshape(1, 4 * HIDDEN)
    b1 = (b_ih1 + b_hh1).reshape(1, 4 * HIDDEN)

    y_tmaj, hN, cN = _lstm_call(e2_pad, par, wih0t, whh0t, b0, wih1t, whh1t,
                                b1, h0, c0)

    y_bmaj = y_tmaj.reshape(SEQ, BATCH, HIDDEN).transpose(1, 0, 2).reshape(
        NTOK, HIDDEN)

    out = _fc_call(y_bmaj, fc_w, fc_b.reshape(1, VOCAB))
    return out, hN, cN
